# baseline (speedup 1.0000x reference)
import functools
import math

import jax
import jax.numpy as jnp
from jax import lax
from jax.experimental import pallas as pl
from jax.experimental.pallas import tpu as pltpu
from jax.experimental.pallas import tpu_sc as plsc

_EMBED = 64
_LANES = 16
_VPR = _EMBED // _LANES
_SCALE = math.sqrt(_EMBED)

_info = plsc.get_sparse_core_info()
_NC, _NS = _info.num_cores, _info.num_subcores
_NW = _NC * _NS

_CHUNK = 128


def _make_lookup(n_idx: int):
    rows_per_w = n_idx // _NW
    n_chunks = rows_per_w // _CHUNK
    mesh = plsc.VectorSubcoreMesh(core_axis_name="c", subcore_axis_name="s")

    @functools.partial(
        pl.kernel,
        out_type=jax.ShapeDtypeStruct((n_idx, _EMBED), jnp.float32),
        mesh=mesh,
        scratch_types=[
            pltpu.VMEM((rows_per_w,), jnp.int32),
            pltpu.VMEM((_CHUNK, _EMBED), jnp.float32),
            pltpu.SemaphoreType.DMA,
        ],
        compiler_params=pltpu.CompilerParams(use_tc_tiling_on_sc=False),
    )
    def lookup(tok_hbm, table_hbm, out_hbm, idx_v, row_v, sem):
        wid = lax.axis_index("s") * _NC + lax.axis_index("c")
        base = wid * rows_per_w
        pltpu.sync_copy(tok_hbm.at[pl.ds(base, rows_per_w)], idx_v)

        @pl.loop(0, n_chunks)
        def _chunk_body(c):
            off = c * _CHUNK
            pltpu.async_copy(
                table_hbm.at[idx_v.at[pl.ds(off, _CHUNK)]], row_v, sem
            ).wait()

            @pl.loop(0, _CHUNK)
            def _scale_body(r):
                for j in range(_VPR):
                    sl = pl.ds(j * _LANES, _LANES)
                    row_v[r, sl] = row_v[r, sl] * _SCALE

            pltpu.sync_copy(row_v, out_hbm.at[pl.ds(base + off, _CHUNK)])

    return lookup


def kernel(tokens, table):
    tok_flat = tokens.reshape(-1).astype(jnp.int32)
    out = _make_lookup(tok_flat.shape[0])(tok_flat, table)
    return out.reshape(tokens.shape + (_EMBED,))


# speedup vs baseline: 1.2082x; 1.2082x over previous
import functools
import math

import jax
import jax.numpy as jnp
from jax import lax
from jax.experimental import pallas as pl
from jax.experimental.pallas import tpu as pltpu
from jax.experimental.pallas import tpu_sc as plsc

_EMBED = 64
_LANES = 16
_VPR = _EMBED // _LANES
_SCALE = math.sqrt(_EMBED)

_info = plsc.get_sparse_core_info()
_NC, _NS = _info.num_cores, _info.num_subcores
_NW = _NC * _NS

_STREAM = 128
_SPC = 2
_CHUNK = _STREAM * _SPC
_NBUF = 2


def _make_lookup(n_idx: int):
    rows_per_w = n_idx // _NW
    n_chunks = rows_per_w // _CHUNK
    mesh = plsc.VectorSubcoreMesh(core_axis_name="c", subcore_axis_name="s")

    @functools.partial(
        pl.kernel,
        out_type=jax.ShapeDtypeStruct((n_idx, _EMBED), jnp.float32),
        mesh=mesh,
        scratch_types=[
            pltpu.VMEM((rows_per_w,), jnp.int32),
            [pltpu.VMEM((_CHUNK, _EMBED), jnp.float32) for _ in range(_NBUF)],
            [pltpu.VMEM((_CHUNK, _EMBED), jnp.float32) for _ in range(_NBUF)],
            [pltpu.SemaphoreType.DMA for _ in range(_NBUF)],
            [pltpu.SemaphoreType.DMA for _ in range(_NBUF)],
        ],
        compiler_params=pltpu.CompilerParams(use_tc_tiling_on_sc=False),
    )
    def lookup(tok_hbm, table_hbm, out_hbm, idx_v, in_bufs, out_bufs, gsems, ssems):
        wid = lax.axis_index("s") * _NC + lax.axis_index("c")
        base = wid * rows_per_w
        pltpu.sync_copy(tok_hbm.at[pl.ds(base, rows_per_w)], idx_v)

        def fire_gathers(c, b):
            for k in range(_SPC):
                off = c * _CHUNK + k * _STREAM
                pltpu.async_copy(
                    table_hbm.at[idx_v.at[pl.ds(off, _STREAM)]],
                    in_bufs[b].at[pl.ds(k * _STREAM, _STREAM)],
                    gsems[b],
                )

        def wait_gathers(c, b):
            for k in range(_SPC):
                off = c * _CHUNK + k * _STREAM
                pltpu.make_async_copy(
                    table_hbm.at[idx_v.at[pl.ds(off, _STREAM)]],
                    in_bufs[b].at[pl.ds(k * _STREAM, _STREAM)],
                    gsems[b],
                ).wait()

        def scatter_desc(c, b):
            return pltpu.make_async_copy(
                out_bufs[b], out_hbm.at[pl.ds(base + c * _CHUNK, _CHUNK)], ssems[b]
            )

        for b in range(_NBUF):
            fire_gathers(b, b)

        @pl.loop(0, n_chunks, step=_NBUF)
        def _pipeline(c0):
            for b in range(_NBUF):
                c = c0 + b
                wait_gathers(c, b)

                @pl.when(c0 >= _NBUF)
                def _():
                    scatter_desc(c - _NBUF, b).wait()

                @pl.loop(0, _CHUNK)
                def _scale(r):
                    for j in range(_VPR):
                        sl = pl.ds(j * _LANES, _LANES)
                        out_bufs[b][r, sl] = in_bufs[b][r, sl] * _SCALE

                scatter_desc(c, b).start()

                @pl.when(c0 < n_chunks - _NBUF)
                def _():
                    fire_gathers(c + _NBUF, b)

        for b in range(_NBUF):
            scatter_desc(n_chunks - _NBUF + b, b).wait()

    return lookup


def kernel(tokens, table):
    tok_flat = tokens.reshape(-1).astype(jnp.int32)
    out = _make_lookup(tok_flat.shape[0])(tok_flat, table)
    return out.reshape(tokens.shape + (_EMBED,))
